# Initial kernel scaffold; baseline (speedup 1.0000x reference)
import functools

import jax
import jax.numpy as jnp
from jax import lax
from jax.experimental import pallas as pl
from jax.experimental.pallas import tpu as pltpu
from jax.experimental.pallas import tpu_sc as plsc

_N_EMBED = 8192
_EMBED_DIM = 256
_COMMITMENT_COST = 0.25

_TN = 1024
_INT_MAX = jnp.int32(2147483647)


def _argmin_body(x_ref, w_ref, idx_ref, loss_ref, rmin_ref, ridx_ref, acc_ref):
    i = pl.program_id(0)
    j = pl.program_id(1)
    ni = pl.num_programs(0)
    nj = pl.num_programs(1)

    xb = x_ref[0]
    wb = w_ref[...]
    s = xb.shape[-1]

    prod = lax.dot_general(
        wb, xb, (((1,), (0,)), ((), ())), preferred_element_type=jnp.float32
    )
    wnorm = jnp.sum(wb * wb, axis=1, keepdims=True)
    xsq = jnp.sum(xb * xb, axis=0, keepdims=True)
    d = (xsq - 2.0 * prod) + wnorm

    mn = jnp.min(d, axis=0)
    jidx = lax.broadcasted_iota(jnp.int32, (wb.shape[0], s), 0) + j * _TN
    lidx = jnp.min(jnp.where(d == mn[None, :], jidx, _INT_MAX), axis=0)

    @pl.when(j == 0)
    def _():
        rmin_ref[...] = mn
        ridx_ref[...] = lidx

    @pl.when(j > 0)
    def _():
        upd = mn < rmin_ref[...]
        ridx_ref[...] = jnp.where(upd, lidx, ridx_ref[...])
        rmin_ref[...] = jnp.where(upd, mn, rmin_ref[...])

    @pl.when(j == nj - 1)
    def _():
        idx_ref[...] = ridx_ref[...]
        part = jnp.sum(rmin_ref[...])

        @pl.when(i == 0)
        def _():
            acc_ref[0] = part

        @pl.when(i > 0)
        def _():
            acc_ref[0] = acc_ref[0] + part

        @pl.when(i == ni - 1)
        def _():
            loss_ref[0] = acc_ref[0]


def _argmin_call(xr, weight):
    b, c, s = xr.shape
    n = weight.shape[0]
    grid = (b, n // _TN)
    return pl.pallas_call(
        _argmin_body,
        grid=grid,
        in_specs=[
            pl.BlockSpec((1, c, s), lambda i, j: (i, 0, 0)),
            pl.BlockSpec((_TN, c), lambda i, j: (j, 0)),
        ],
        out_specs=[
            pl.BlockSpec((s,), lambda i, j: (i,)),
            pl.BlockSpec(memory_space=pltpu.SMEM),
        ],
        out_shape=[
            jax.ShapeDtypeStruct((b * s,), jnp.int32),
            jax.ShapeDtypeStruct((1,), jnp.float32),
        ],
        scratch_shapes=[
            pltpu.VMEM((s,), jnp.float32),
            pltpu.VMEM((s,), jnp.int32),
            pltpu.SMEM((1,), jnp.float32),
        ],
        compiler_params=pltpu.CompilerParams(
            dimension_semantics=("arbitrary", "arbitrary"),
        ),
    )(xr, weight)


_NC = 2
_NS = 16
_NW = _NC * _NS
_CH = 128


def _gather_rows(weight, idx):
    b_tot = idx.shape[0]
    d = weight.shape[1]
    bpw = b_tot // _NW
    nch = bpw // _CH
    mesh = plsc.VectorSubcoreMesh(core_axis_name="c", subcore_axis_name="s")

    @functools.partial(
        pl.kernel,
        mesh=mesh,
        out_type=jax.ShapeDtypeStruct((b_tot, d), jnp.float32),
        scratch_types=[
            pltpu.VMEM((_CH,), jnp.int32),
            pltpu.VMEM((_CH, d), jnp.float32),
            pltpu.SemaphoreType.DMA,
        ],
    )
    def gk(w_hbm, idx_hbm, out_hbm, idx_v, rows_v, sem):
        wid = lax.axis_index("s") * _NC + lax.axis_index("c")
        base = wid * bpw
        for k in range(nch):
            off = base + k * _CH
            pltpu.sync_copy(idx_hbm.at[pl.ds(off, _CH)], idx_v)
            pltpu.async_copy(w_hbm.at[idx_v], rows_v, sem).wait()
            pltpu.sync_copy(rows_v, out_hbm.at[pl.ds(off, _CH)])

    return gk(weight, idx)


def kernel(x, weight):
    b, c, h, w = x.shape
    s = h * w
    xr = x.reshape(b, c, s)
    idx_flat, loss_sum = _argmin_call(xr, weight)
    qflat = _gather_rows(weight, idx_flat)
    q = qflat.reshape(b, s, c).transpose(0, 2, 1).reshape(b, c, h, w)
    embed_idx = idx_flat.reshape(b, h, w)
    latent_loss = (loss_sum[0] * (_COMMITMENT_COST / (b * c * h * w))).reshape(())
    return (q, embed_idx, latent_loss)


# speedup vs baseline: 1.1088x; 1.1088x over previous
import functools

import jax
import jax.numpy as jnp
from jax import lax
from jax.experimental import pallas as pl
from jax.experimental.pallas import tpu as pltpu
from jax.experimental.pallas import tpu_sc as plsc

_N_EMBED = 8192
_EMBED_DIM = 256
_COMMITMENT_COST = 0.25

_TN = 1024


def _argmin_body(x_ref, w_ref, idx_ref, loss_ref, rmin_ref, ridx_ref, rloss_ref,
                 acc_ref):
    i = pl.program_id(0)
    j = pl.program_id(1)
    ni = pl.num_programs(0)
    nj = pl.num_programs(1)
    half = nj // 2

    xb = x_ref[0]
    wb = w_ref[...]
    s = xb.shape[-1]

    prod = lax.dot_general(
        wb.astype(jnp.bfloat16), xb.astype(jnp.bfloat16),
        (((1,), (0,)), ((), ())),
        preferred_element_type=jnp.float32,
    )
    wnorm = jnp.sum(wb * wb, axis=1, keepdims=True)
    xsq = jnp.sum(xb * xb, axis=0, keepdims=True)
    d = (xsq - 2.0 * prod) + wnorm

    mn = jnp.min(d, axis=0)
    jidx = lax.broadcasted_iota(jnp.int32, (wb.shape[0], s), 0) + j * _TN
    lidx = jnp.min(jnp.where(d == mn[None, :], jidx, jnp.int32(2147483647)), axis=0)

    @pl.when(j == 0)
    def _():
        rmin_ref[...] = mn
        ridx_ref[...] = lidx
        rloss_ref[...] = mn

    @pl.when(jnp.logical_and(j > 0, j != half))
    def _():
        upd = mn < rmin_ref[...]
        ridx_ref[...] = jnp.where(upd, lidx, ridx_ref[...])
        rmin_ref[...] = jnp.where(upd, mn, rmin_ref[...])
        rloss_ref[...] = jnp.where(upd, mn, rloss_ref[...])

    @pl.when(j == half)
    def _():
        q = rmin_ref[...].astype(jnp.bfloat16).astype(jnp.float32)
        upd = mn < q
        ridx_ref[...] = jnp.where(upd, lidx, ridx_ref[...])
        rmin_ref[...] = jnp.where(upd, mn, q)
        rloss_ref[...] = jnp.where(upd, mn, rloss_ref[...])

    @pl.when(j == nj - 1)
    def _():
        idx_ref[...] = ridx_ref[...]
        part = jnp.sum(rloss_ref[...])

        @pl.when(i == 0)
        def _():
            acc_ref[0] = part

        @pl.when(i > 0)
        def _():
            acc_ref[0] = acc_ref[0] + part

        @pl.when(i == ni - 1)
        def _():
            loss_ref[0] = acc_ref[0]


def _argmin_call(xr, weight):
    b, c, s = xr.shape
    n = weight.shape[0]
    grid = (b, n // _TN)
    return pl.pallas_call(
        _argmin_body,
        grid=grid,
        in_specs=[
            pl.BlockSpec((1, c, s), lambda i, j: (i, 0, 0)),
            pl.BlockSpec((_TN, c), lambda i, j: (j, 0)),
        ],
        out_specs=[
            pl.BlockSpec((s,), lambda i, j: (i,)),
            pl.BlockSpec(memory_space=pltpu.SMEM),
        ],
        out_shape=[
            jax.ShapeDtypeStruct((b * s,), jnp.int32),
            jax.ShapeDtypeStruct((1,), jnp.float32),
        ],
        scratch_shapes=[
            pltpu.VMEM((s,), jnp.float32),
            pltpu.VMEM((s,), jnp.int32),
            pltpu.VMEM((s,), jnp.float32),
            pltpu.SMEM((1,), jnp.float32),
        ],
        compiler_params=pltpu.CompilerParams(
            dimension_semantics=("arbitrary", "arbitrary"),
        ),
    )(xr, weight)


_NC = 2
_NS = 16
_NW = _NC * _NS
_CH = 128


def _gather_rows(weight, idx):
    b_tot = idx.shape[0]
    d = weight.shape[1]
    bpw = b_tot // _NW
    nch = bpw // _CH
    mesh = plsc.VectorSubcoreMesh(core_axis_name="c", subcore_axis_name="s")

    @functools.partial(
        pl.kernel,
        mesh=mesh,
        out_type=jax.ShapeDtypeStruct((b_tot, d), jnp.float32),
        scratch_types=[
            pltpu.VMEM((_CH,), jnp.int32),
            pltpu.VMEM((_CH, d), jnp.float32),
            pltpu.SemaphoreType.DMA,
        ],
    )
    def gk(w_hbm, idx_hbm, out_hbm, idx_v, rows_v, sem):
        wid = lax.axis_index("s") * _NC + lax.axis_index("c")
        base = wid * bpw
        for k in range(nch):
            off = base + k * _CH
            pltpu.sync_copy(idx_hbm.at[pl.ds(off, _CH)], idx_v)
            pltpu.async_copy(w_hbm.at[idx_v], rows_v, sem).wait()
            pltpu.sync_copy(rows_v, out_hbm.at[pl.ds(off, _CH)])

    return gk(weight, idx)


def kernel(x, weight):
    b, c, h, w = x.shape
    s = h * w
    xr = x.reshape(b, c, s)
    idx_flat, loss_sum = _argmin_call(xr, weight)
    qflat = _gather_rows(weight, idx_flat)
    q = qflat.reshape(b, s, c).transpose(0, 2, 1).reshape(b, c, h, w)
    embed_idx = idx_flat.reshape(b, h, w)
    latent_loss = (loss_sum[0] * (_COMMITMENT_COST / (b * c * h * w))).reshape(())
    return (q, embed_idx, latent_loss)


# speedup vs baseline: 1.1894x; 1.0726x over previous
import functools

import jax
import jax.numpy as jnp
from jax import lax
from jax.experimental import pallas as pl
from jax.experimental.pallas import tpu as pltpu
from jax.experimental.pallas import tpu_sc as plsc

_N_EMBED = 8192
_EMBED_DIM = 256
_COMMITMENT_COST = 0.25

_TN = 2048


def _argmin_body(x_ref, w_ref, idx_ref, loss_ref, rmin_ref, ridx_ref, rloss_ref,
                 acc_ref):
    i = pl.program_id(0)
    j = pl.program_id(1)
    ni = pl.num_programs(0)
    nj = pl.num_programs(1)
    half = nj // 2

    xb = x_ref[0]
    wb = w_ref[...]
    s = xb.shape[-1]

    prod = lax.dot_general(
        wb.astype(jnp.bfloat16), xb.astype(jnp.bfloat16),
        (((1,), (0,)), ((), ())),
        preferred_element_type=jnp.float32,
    )
    wnorm = jnp.sum(wb * wb, axis=1, keepdims=True)
    xsq = jnp.sum(xb * xb, axis=0, keepdims=True)
    d = (xsq - 2.0 * prod) + wnorm

    mn = jnp.min(d, axis=0)
    jidx = lax.broadcasted_iota(jnp.int32, (wb.shape[0], s), 0) + j * _TN
    lidx = jnp.min(jnp.where(d == mn[None, :], jidx, jnp.int32(2147483647)), axis=0)

    @pl.when(j == 0)
    def _():
        rmin_ref[...] = mn
        ridx_ref[...] = lidx
        rloss_ref[...] = mn

    @pl.when(jnp.logical_and(j > 0, j != half))
    def _():
        upd = mn < rmin_ref[...]
        ridx_ref[...] = jnp.where(upd, lidx, ridx_ref[...])
        rmin_ref[...] = jnp.where(upd, mn, rmin_ref[...])
        rloss_ref[...] = jnp.where(upd, mn, rloss_ref[...])

    @pl.when(j == half)
    def _():
        q = rmin_ref[...].astype(jnp.bfloat16).astype(jnp.float32)
        upd = mn < q
        ridx_ref[...] = jnp.where(upd, lidx, ridx_ref[...])
        rmin_ref[...] = jnp.where(upd, mn, q)
        rloss_ref[...] = jnp.where(upd, mn, rloss_ref[...])

    @pl.when(j == nj - 1)
    def _():
        idx_ref[...] = ridx_ref[...]
        part = jnp.sum(rloss_ref[...])

        @pl.when(i == 0)
        def _():
            acc_ref[0] = part

        @pl.when(i > 0)
        def _():
            acc_ref[0] = acc_ref[0] + part

        @pl.when(i == ni - 1)
        def _():
            loss_ref[0] = acc_ref[0]


def _argmin_call(xr, weight):
    b, c, s = xr.shape
    n = weight.shape[0]
    grid = (b, n // _TN)
    return pl.pallas_call(
        _argmin_body,
        grid=grid,
        in_specs=[
            pl.BlockSpec((1, c, s), lambda i, j: (i, 0, 0)),
            pl.BlockSpec((_TN, c), lambda i, j: (j, 0)),
        ],
        out_specs=[
            pl.BlockSpec((s,), lambda i, j: (i,)),
            pl.BlockSpec(memory_space=pltpu.SMEM),
        ],
        out_shape=[
            jax.ShapeDtypeStruct((b * s,), jnp.int32),
            jax.ShapeDtypeStruct((1,), jnp.float32),
        ],
        scratch_shapes=[
            pltpu.VMEM((s,), jnp.float32),
            pltpu.VMEM((s,), jnp.int32),
            pltpu.VMEM((s,), jnp.float32),
            pltpu.SMEM((1,), jnp.float32),
        ],
        compiler_params=pltpu.CompilerParams(
            dimension_semantics=("arbitrary", "arbitrary"),
        ),
    )(xr, weight)


_NC = 2
_NS = 16
_NW = _NC * _NS
_CH = 128


def _gather_rows(weight, idx):
    b_tot = idx.shape[0]
    d = weight.shape[1]
    bpw = b_tot // _NW
    nch = bpw // _CH
    mesh = plsc.VectorSubcoreMesh(core_axis_name="c", subcore_axis_name="s")

    @functools.partial(
        pl.kernel,
        mesh=mesh,
        out_type=jax.ShapeDtypeStruct((b_tot, d), jnp.float32),
        scratch_types=[
            pltpu.VMEM((_CH,), jnp.int32),
            pltpu.VMEM((_CH,), jnp.int32),
            pltpu.VMEM((_CH, d), jnp.float32),
            pltpu.VMEM((_CH, d), jnp.float32),
            pltpu.SemaphoreType.DMA,
            pltpu.SemaphoreType.DMA,
        ],
    )
    def gk(w_hbm, idx_hbm, out_hbm, iv0, iv1, rv0, rv1, s0, s1):
        wid = lax.axis_index("s") * _NC + lax.axis_index("c")
        base = wid * bpw
        ivs, rvs, sems = [iv0, iv1], [rv0, rv1], [s0, s1]
        cps = [None] * nch
        for k in range(nch):
            if k == 0:
                pltpu.sync_copy(idx_hbm.at[pl.ds(base, _CH)], ivs[0])
                cps[0] = pltpu.async_copy(w_hbm.at[ivs[0]], rvs[0], sems[0])
            if k + 1 < nch:
                nb = (k + 1) % 2
                pltpu.sync_copy(idx_hbm.at[pl.ds(base + (k + 1) * _CH, _CH)], ivs[nb])
                cps[k + 1] = pltpu.async_copy(w_hbm.at[ivs[nb]], rvs[nb], sems[nb])
            cps[k].wait()
            pltpu.sync_copy(rvs[k % 2], out_hbm.at[pl.ds(base + k * _CH, _CH)])

    return gk(weight, idx)


def kernel(x, weight):
    b, c, h, w = x.shape
    s = h * w
    xr = x.reshape(b, c, s)
    idx_flat, loss_sum = _argmin_call(xr, weight)
    qflat = _gather_rows(weight, idx_flat)
    q = qflat.reshape(b, s, c).transpose(0, 2, 1).reshape(b, c, h, w)
    embed_idx = idx_flat.reshape(b, h, w)
    latent_loss = (loss_sum[0] * (_COMMITMENT_COST / (b * c * h * w))).reshape(())
    return (q, embed_idx, latent_loss)


# speedup vs baseline: 1.2464x; 1.0480x over previous
import functools

import jax
import jax.numpy as jnp
from jax import lax
from jax.experimental import pallas as pl
from jax.experimental.pallas import tpu as pltpu
from jax.experimental.pallas import tpu_sc as plsc

_N_EMBED = 8192
_EMBED_DIM = 256
_COMMITMENT_COST = 0.25

_TN = 4096


def _argmin_body(x_ref, w_ref, idx_ref, loss_ref, rmin_ref, ridx_ref, rloss_ref,
                 acc_ref):
    i = pl.program_id(0)
    j = pl.program_id(1)
    ni = pl.num_programs(0)
    nj = pl.num_programs(1)
    half = nj // 2

    xb = x_ref[0]
    wb = w_ref[...]
    s = xb.shape[-1]

    prod = lax.dot_general(
        wb.astype(jnp.bfloat16), xb.astype(jnp.bfloat16),
        (((1,), (0,)), ((), ())),
        preferred_element_type=jnp.float32,
    )
    wnorm = jnp.sum(wb * wb, axis=1, keepdims=True)
    xsq = jnp.sum(xb * xb, axis=0, keepdims=True)
    d = (xsq - 2.0 * prod) + wnorm

    mn = jnp.min(d, axis=0)
    jidx = lax.broadcasted_iota(jnp.int32, (wb.shape[0], s), 0) + j * _TN
    lidx = jnp.min(jnp.where(d == mn[None, :], jidx, jnp.int32(2147483647)), axis=0)

    @pl.when(j == 0)
    def _():
        rmin_ref[...] = mn
        ridx_ref[...] = lidx
        rloss_ref[...] = mn

    @pl.when(jnp.logical_and(j > 0, j != half))
    def _():
        upd = mn < rmin_ref[...]
        ridx_ref[...] = jnp.where(upd, lidx, ridx_ref[...])
        rmin_ref[...] = jnp.where(upd, mn, rmin_ref[...])
        rloss_ref[...] = jnp.where(upd, mn, rloss_ref[...])

    @pl.when(j == half)
    def _():
        q = rmin_ref[...].astype(jnp.bfloat16).astype(jnp.float32)
        upd = mn < q
        ridx_ref[...] = jnp.where(upd, lidx, ridx_ref[...])
        rmin_ref[...] = jnp.where(upd, mn, q)
        rloss_ref[...] = jnp.where(upd, mn, rloss_ref[...])

    @pl.when(j == nj - 1)
    def _():
        idx_ref[...] = ridx_ref[...]
        part = jnp.sum(rloss_ref[...])

        @pl.when(i == 0)
        def _():
            acc_ref[0] = part

        @pl.when(i > 0)
        def _():
            acc_ref[0] = acc_ref[0] + part

        @pl.when(i == ni - 1)
        def _():
            loss_ref[0] = acc_ref[0]


def _argmin_call(xr, weight):
    b, c, s = xr.shape
    n = weight.shape[0]
    grid = (b, n // _TN)
    return pl.pallas_call(
        _argmin_body,
        grid=grid,
        in_specs=[
            pl.BlockSpec((1, c, s), lambda i, j: (i, 0, 0)),
            pl.BlockSpec((_TN, c), lambda i, j: (j, 0)),
        ],
        out_specs=[
            pl.BlockSpec((s,), lambda i, j: (i,)),
            pl.BlockSpec(memory_space=pltpu.SMEM),
        ],
        out_shape=[
            jax.ShapeDtypeStruct((b * s,), jnp.int32),
            jax.ShapeDtypeStruct((1,), jnp.float32),
        ],
        scratch_shapes=[
            pltpu.VMEM((s,), jnp.float32),
            pltpu.VMEM((s,), jnp.int32),
            pltpu.VMEM((s,), jnp.float32),
            pltpu.SMEM((1,), jnp.float32),
        ],
        compiler_params=pltpu.CompilerParams(
            dimension_semantics=("arbitrary", "arbitrary"),
        ),
    )(xr, weight)


_NC = 2
_NS = 16
_NW = _NC * _NS
_CH = 128


def _gather_rows(weight, idx):
    b_tot = idx.shape[0]
    d = weight.shape[1]
    bpw = b_tot // _NW
    nch = bpw // _CH
    mesh = plsc.VectorSubcoreMesh(core_axis_name="c", subcore_axis_name="s")

    @functools.partial(
        pl.kernel,
        mesh=mesh,
        out_type=jax.ShapeDtypeStruct((b_tot, d), jnp.float32),
        scratch_types=[
            pltpu.VMEM((_CH,), jnp.int32),
            pltpu.VMEM((_CH,), jnp.int32),
            pltpu.VMEM((_CH, d), jnp.float32),
            pltpu.VMEM((_CH, d), jnp.float32),
            pltpu.SemaphoreType.DMA,
            pltpu.SemaphoreType.DMA,
        ],
    )
    def gk(w_hbm, idx_hbm, out_hbm, iv0, iv1, rv0, rv1, s0, s1):
        wid = lax.axis_index("s") * _NC + lax.axis_index("c")
        base = wid * bpw
        ivs, rvs, sems = [iv0, iv1], [rv0, rv1], [s0, s1]
        cps = [None] * nch
        for k in range(nch):
            if k == 0:
                pltpu.sync_copy(idx_hbm.at[pl.ds(base, _CH)], ivs[0])
                cps[0] = pltpu.async_copy(w_hbm.at[ivs[0]], rvs[0], sems[0])
            if k + 1 < nch:
                nb = (k + 1) % 2
                pltpu.sync_copy(idx_hbm.at[pl.ds(base + (k + 1) * _CH, _CH)], ivs[nb])
                cps[k + 1] = pltpu.async_copy(w_hbm.at[ivs[nb]], rvs[nb], sems[nb])
            cps[k].wait()
            pltpu.sync_copy(rvs[k % 2], out_hbm.at[pl.ds(base + k * _CH, _CH)])

    return gk(weight, idx)


def kernel(x, weight):
    b, c, h, w = x.shape
    s = h * w
    xr = x.reshape(b, c, s)
    idx_flat, loss_sum = _argmin_call(xr, weight)
    qflat = _gather_rows(weight, idx_flat)
    q = qflat.reshape(b, s, c).transpose(0, 2, 1).reshape(b, c, h, w)
    embed_idx = idx_flat.reshape(b, h, w)
    latent_loss = (loss_sum[0] * (_COMMITMENT_COST / (b * c * h * w))).reshape(())
    return (q, embed_idx, latent_loss)
